# Initial kernel scaffold; baseline (speedup 1.0000x reference)
import jax
import jax.numpy as jnp
from jax.experimental import pallas as pl


def kernel(x, edge_index, scope, emb, W, b, scale):
    raise NotImplementedError("write your pallas kernel here")


# speedup vs baseline: 6.1655x; 6.1655x over previous
import functools

import jax
import jax.numpy as jnp
from jax import lax
from jax.experimental import pallas as pl
from jax.experimental.pallas import tpu as pltpu
from jax.experimental.pallas import tpu_sc as plsc

N_NODES = 10000
N_EDGES = 160000
D = 256
HALF = 128
VOCAB = 780
N_TREES = 100
TREE = 100

NC = 2
NS = 16
NW = NC * NS


_EPAD = 10240
_ECHUNK = 80
_ECHUNKS = 4


def _embed_gather(x, emb):
    xp = jnp.concatenate([x, jnp.zeros((_EPAD - N_NODES,), jnp.int32)])
    x3 = xp.reshape(NW, _ECHUNKS, _ECHUNK)
    mesh = plsc.VectorSubcoreMesh(core_axis_name="c", subcore_axis_name="s")

    @functools.partial(
        pl.kernel,
        out_type=jax.ShapeDtypeStruct((_EPAD, D), jnp.float32),
        mesh=mesh,
        scratch_types=[
            pltpu.VMEM((_ECHUNKS, _ECHUNK), jnp.int32),
            pltpu.VMEM((_ECHUNK, D), jnp.float32),
            pltpu.VMEM((_ECHUNK, D), jnp.float32),
            pltpu.SemaphoreType.DMA,
            pltpu.SemaphoreType.DMA,
        ],
    )
    def k(emb_hbm, x_hbm, out_hbm, idx_v, rows0, rows1, semA, semB):
        c = lax.axis_index("c")
        s = lax.axis_index("s")
        w = s * NC + c
        pltpu.sync_copy(x_hbm.at[w], idx_v)
        base = w * (_ECHUNKS * _ECHUNK)
        pltpu.make_async_copy(emb_hbm.at[idx_v.at[0]], rows0, semA).start()
        pltpu.make_async_copy(emb_hbm.at[idx_v.at[1]], rows1, semB).start()
        for j in range(_ECHUNKS):
            buf = rows0 if j % 2 == 0 else rows1
            sem = semA if j % 2 == 0 else semB
            pltpu.make_async_copy(emb_hbm.at[idx_v.at[j]], buf, sem).wait()
            pltpu.sync_copy(buf, out_hbm.at[pl.ds(base + j * _ECHUNK, _ECHUNK)])
            if j + 2 < _ECHUNKS:
                pltpu.make_async_copy(emb_hbm.at[idx_v.at[j + 2]], buf, sem).start()

    return k(emb, x3)


def _lorentz_layer(h_full, halves, Wi, bi, scalei, nonlin):
    if h_full is not None:
        R = h_full.shape[0]
    else:
        R = halves[0].shape[0]
    BLK = 1024 if R % 1024 == 0 else 1000
    grid = R // BLK
    b2 = jnp.broadcast_to(bi.reshape(1, D), (8, D))
    s2 = jnp.broadcast_to(scalei.reshape(1, 1), (8, D))

    def body(*refs):
        if h_full is not None:
            h_ref, w_ref, b_ref, sc_ref, o0_ref, o1_ref = refs
            h = h_ref[...]
        else:
            s0_ref, s1_ref, w_ref, b_ref, sc_ref, o0_ref, o1_ref = refs
            h = jnp.concatenate([s0_ref[...], s1_ref[...]], axis=-1)
            col = lax.broadcasted_iota(jnp.int32, h.shape, 1)
            is0 = col == 0
            t = jnp.sum(jnp.where(is0, h, 0.0), axis=-1, keepdims=True)
            nsq = jnp.sum(jnp.where(is0, 0.0, h * h), axis=-1, keepdims=True)
            denom = jnp.sqrt(jnp.clip(jnp.abs(t * t - nsq), 1e-8, None))
            h = h / denom
        if nonlin:
            h = jnp.maximum(h, 0.0)
        y = lax.dot_general(h, w_ref[...], (((1,), (1,)), ((), ())),
                            precision=lax.Precision.HIGHEST,
                            preferred_element_type=jnp.float32)
        y = y + b_ref[0:1, :]
        col = lax.broadcasted_iota(jnp.int32, y.shape, 1)
        is0 = col == 0
        y0 = jnp.sum(jnp.where(is0, y, 0.0), axis=-1, keepdims=True)
        nsq = jnp.sum(jnp.where(is0, 0.0, y * y), axis=-1, keepdims=True)
        es = jnp.exp(sc_ref[0:1, 0:1])
        time = 1.0 / (1.0 + jnp.exp(-y0)) * es + 1.1
        sfac = (time * time - 1.0) / jnp.clip(nsq, 1e-8, None)
        out = jnp.where(is0, time, y * jnp.sqrt(sfac))
        o0_ref[...] = out[:, :HALF]
        o1_ref[...] = out[:, HALF:]

    if h_full is not None:
        in_arrays = (h_full,)
        in_specs = [pl.BlockSpec((BLK, D), lambda i: (i, 0))]
    else:
        in_arrays = halves
        in_specs = [pl.BlockSpec((BLK, HALF), lambda i: (i, 0)),
                    pl.BlockSpec((BLK, HALF), lambda i: (i, 0))]
    in_specs += [pl.BlockSpec((D, D), lambda i: (0, 0)),
                 pl.BlockSpec((8, D), lambda i: (0, 0)),
                 pl.BlockSpec((8, D), lambda i: (0, 0))]
    out_specs = [pl.BlockSpec((BLK, HALF), lambda i: (i, 0)),
                 pl.BlockSpec((BLK, HALF), lambda i: (i, 0))]
    return pl.pallas_call(
        body,
        grid=(grid,),
        in_specs=in_specs,
        out_specs=out_specs,
        out_shape=[jax.ShapeDtypeStruct((R, HALF), jnp.float32)] * 2,
    )(*in_arrays, Wi, b2, s2)


_CHUNK = 100
_NPHASE = 2
_PCHUNK = 50
_RPAD = 10240
_SLAB = _RPAD // NS


def _edge_agg(h0, h1, src3, dst3):
    mesh = plsc.VectorSubcoreMesh(core_axis_name="c", subcore_axis_name="s")
    z = jnp.zeros((_RPAD, HALF), jnp.float32)

    @functools.partial(
        pl.kernel,
        out_type=(jax.ShapeDtypeStruct((_RPAD, HALF), jnp.float32),
                  jax.ShapeDtypeStruct((_RPAD, HALF), jnp.float32)),
        mesh=mesh,
        scratch_types=[
            pltpu.VMEM((_PCHUNK, _CHUNK), jnp.int32),
            pltpu.VMEM((_PCHUNK, _CHUNK), jnp.int32),
            pltpu.VMEM((_CHUNK, HALF), jnp.float32),
            pltpu.VMEM((_CHUNK, HALF), jnp.float32),
            pltpu.VMEM_SHARED((_RPAD, HALF), jnp.float32),
            pltpu.SemaphoreType.DMA,
            pltpu.SemaphoreType.DMA,
        ],
    )
    def k(h0_hbm, h1_hbm, src_hbm, dst_hbm, z_hbm, o0_hbm, o1_hbm,
          src_v, dst_v, buf0, buf1, acc, semA, semB):
        c = lax.axis_index("c")
        s = lax.axis_index("s")

        def run(h_hbm, o_hbm):
            pltpu.sync_copy(z_hbm.at[pl.ds(s * _SLAB, _SLAB)],
                            acc.at[pl.ds(s * _SLAB, _SLAB)])
            plsc.subcore_barrier()
            for p in range(_NPHASE):
                pltpu.sync_copy(src_hbm.at[s, p], src_v)
                pltpu.sync_copy(dst_hbm.at[s, p], dst_v)
                pltpu.make_async_copy(h_hbm.at[src_v.at[0]], buf0, semA).start()

                def body(kk, carry):
                    j0 = 2 * kk
                    pltpu.make_async_copy(h_hbm.at[src_v.at[j0 + 1]], buf1, semB).start()
                    pltpu.make_async_copy(h_hbm.at[src_v.at[j0]], buf0, semA).wait()
                    pltpu.sync_copy(buf0, acc.at[dst_v.at[j0]], add=True)
                    j2 = lax.rem(j0 + 2, _PCHUNK)
                    pltpu.make_async_copy(h_hbm.at[src_v.at[j2]], buf0, semA).start()
                    pltpu.make_async_copy(h_hbm.at[src_v.at[j0 + 1]], buf1, semB).wait()
                    pltpu.sync_copy(buf1, acc.at[dst_v.at[j0 + 1]], add=True)
                    return carry

                lax.fori_loop(0, _PCHUNK // 2, body, 0)
                pltpu.make_async_copy(h_hbm.at[src_v.at[0]], buf0, semA).wait()
            plsc.subcore_barrier()
            pltpu.sync_copy(acc.at[pl.ds(s * _SLAB, _SLAB)],
                            o_hbm.at[pl.ds(s * _SLAB, _SLAB)])

        @pl.when(c == 0)
        def _():
            run(h0_hbm, o0_hbm)

        @pl.when(c == 1)
        def _():
            run(h1_hbm, o1_hbm)

    return k(h0, h1, src3, dst3, z)


_FBLK = 500
_FTREES = 5
_FGRID = N_NODES // _FBLK


def _finalize(s0, s1, scope):
    s0_3 = s0[:N_NODES].reshape(_FGRID, _FBLK, HALF)
    s1_3 = s1[:N_NODES].reshape(_FGRID, _FBLK, HALF)
    scope3 = jnp.broadcast_to(
        scope.astype(jnp.float32).reshape(_FGRID, _FTREES, 1),
        (_FGRID, _FTREES, D))

    def body(s0_ref, s1_ref, sc_ref, h_ref, t_ref):
        h = jnp.concatenate([s0_ref[0], s1_ref[0]], axis=-1)
        col = lax.broadcasted_iota(jnp.int32, h.shape, 1)
        is0 = col == 0
        t = jnp.sum(jnp.where(is0, h, 0.0), axis=-1, keepdims=True)
        nsq = jnp.sum(jnp.where(is0, 0.0, h * h), axis=-1, keepdims=True)
        denom = jnp.sqrt(jnp.clip(jnp.abs(t * t - nsq), 1e-8, None))
        hn = h / denom
        h_ref[...] = hn[None]
        ti = lax.broadcasted_iota(jnp.int32, (_FTREES, _FBLK), 0)
        ri = lax.broadcasted_iota(jnp.int32, (_FTREES, _FBLK), 1)
        mask = (ri // TREE == ti).astype(jnp.float32)
        sums = jnp.dot(mask, hn, precision=lax.Precision.HIGHEST,
                       preferred_element_type=jnp.float32)
        ave = sums / sc_ref[0]
        col5 = lax.broadcasted_iota(jnp.int32, ave.shape, 1)
        is05 = col5 == 0
        t5 = jnp.sum(jnp.where(is05, ave, 0.0), axis=-1, keepdims=True)
        nsq5 = jnp.sum(jnp.where(is05, 0.0, ave * ave), axis=-1, keepdims=True)
        denom5 = jnp.sqrt(jnp.clip(jnp.abs(t5 * t5 - nsq5), 1e-8, None))
        t_ref[...] = (ave / denom5)[None]

    h3, tree3 = pl.pallas_call(
        body,
        grid=(_FGRID,),
        in_specs=[pl.BlockSpec((1, _FBLK, HALF), lambda i: (i, 0, 0)),
                  pl.BlockSpec((1, _FBLK, HALF), lambda i: (i, 0, 0)),
                  pl.BlockSpec((1, _FTREES, D), lambda i: (i, 0, 0))],
        out_specs=[pl.BlockSpec((1, _FBLK, D), lambda i: (i, 0, 0)),
                   pl.BlockSpec((1, _FTREES, D), lambda i: (i, 0, 0))],
        out_shape=[jax.ShapeDtypeStruct((_FGRID, _FBLK, D), jnp.float32),
                   jax.ShapeDtypeStruct((_FGRID, _FTREES, D), jnp.float32)],
    )(s0_3, s1_3, scope3)
    return tree3.reshape(N_TREES, D), h3.reshape(N_NODES, D)


def kernel(x, edge_index, scope, emb, W, b, scale):
    x = x.astype(jnp.int32)
    src = edge_index[0].astype(jnp.int32)
    dst = edge_index[1].astype(jnp.int32)
    src3 = src.reshape(NS, _NPHASE, _PCHUNK, _CHUNK)
    dst3 = dst.reshape(NS, _NPHASE, _PCHUNK, _CHUNK)

    h_emb = _embed_gather(x, emb)
    h0, h1 = _lorentz_layer(h_emb, None, W[0], b[0], scale[0], nonlin=False)
    s0, s1 = _edge_agg(h0, h1, src3, dst3)
    h0, h1 = _lorentz_layer(None, (s0, s1), W[1], b[1], scale[1], nonlin=True)
    s0, s1 = _edge_agg(h0, h1, src3, dst3)
    h0, h1 = _lorentz_layer(None, (s0, s1), W[2], b[2], scale[2], nonlin=True)
    s0, s1 = _edge_agg(h0, h1, src3, dst3)
    tree_vecs, h = _finalize(s0, s1, scope)
    return (tree_vecs, h)


# speedup vs baseline: 6.3258x; 1.0260x over previous
import functools

import jax
import jax.numpy as jnp
from jax import lax
from jax.experimental import pallas as pl
from jax.experimental.pallas import tpu as pltpu
from jax.experimental.pallas import tpu_sc as plsc

N_NODES = 10000
N_EDGES = 160000
D = 256
HALF = 128
VOCAB = 780
N_TREES = 100
TREE = 100

NC = 2
NS = 16
NW = NC * NS


_EPAD = 10240
_ECHUNK = 80
_ECHUNKS = 4


def _embed_gather(x, emb):
    xp = jnp.concatenate([x, jnp.zeros((_EPAD - N_NODES,), jnp.int32)])
    x3 = xp.reshape(NW, _ECHUNKS, _ECHUNK)
    mesh = plsc.VectorSubcoreMesh(core_axis_name="c", subcore_axis_name="s")

    @functools.partial(
        pl.kernel,
        out_type=jax.ShapeDtypeStruct((_EPAD, D), jnp.float32),
        mesh=mesh,
        scratch_types=[
            pltpu.VMEM((_ECHUNKS, _ECHUNK), jnp.int32),
            pltpu.VMEM((_ECHUNK, D), jnp.float32),
            pltpu.VMEM((_ECHUNK, D), jnp.float32),
            pltpu.SemaphoreType.DMA,
            pltpu.SemaphoreType.DMA,
        ],
    )
    def k(emb_hbm, x_hbm, out_hbm, idx_v, rows0, rows1, semA, semB):
        c = lax.axis_index("c")
        s = lax.axis_index("s")
        w = s * NC + c
        pltpu.sync_copy(x_hbm.at[w], idx_v)
        base = w * (_ECHUNKS * _ECHUNK)
        pltpu.make_async_copy(emb_hbm.at[idx_v.at[0]], rows0, semA).start()
        pltpu.make_async_copy(emb_hbm.at[idx_v.at[1]], rows1, semB).start()
        for j in range(_ECHUNKS):
            buf = rows0 if j % 2 == 0 else rows1
            sem = semA if j % 2 == 0 else semB
            pltpu.make_async_copy(emb_hbm.at[idx_v.at[j]], buf, sem).wait()
            pltpu.sync_copy(buf, out_hbm.at[pl.ds(base + j * _ECHUNK, _ECHUNK)])
            if j + 2 < _ECHUNKS:
                pltpu.make_async_copy(emb_hbm.at[idx_v.at[j + 2]], buf, sem).start()

    return k(emb, x3)


def _lorentz_layer(h_full, halves, Wi, bi, scalei, nonlin):
    if h_full is not None:
        R = h_full.shape[0]
    else:
        R = halves[0].shape[0]
    BLK = 1024 if R % 1024 == 0 else 1000
    grid = R // BLK
    b2 = jnp.broadcast_to(bi.reshape(1, D), (8, D))
    s2 = jnp.broadcast_to(scalei.reshape(1, 1), (8, D))

    def body(*refs):
        if h_full is not None:
            h_ref, w_ref, b_ref, sc_ref, o0_ref, o1_ref = refs
            h = h_ref[...]
        else:
            s0_ref, s1_ref, w_ref, b_ref, sc_ref, o0_ref, o1_ref = refs
            h = jnp.concatenate([s0_ref[...], s1_ref[...]], axis=-1)
            col = lax.broadcasted_iota(jnp.int32, h.shape, 1)
            is0 = col == 0
            t = jnp.sum(jnp.where(is0, h, 0.0), axis=-1, keepdims=True)
            nsq = jnp.sum(jnp.where(is0, 0.0, h * h), axis=-1, keepdims=True)
            denom = jnp.sqrt(jnp.clip(jnp.abs(t * t - nsq), 1e-8, None))
            h = h / denom
        if nonlin:
            h = jnp.maximum(h, 0.0)
        y = lax.dot_general(h, w_ref[...], (((1,), (1,)), ((), ())),
                            precision=lax.Precision.HIGHEST,
                            preferred_element_type=jnp.float32)
        y = y + b_ref[0:1, :]
        col = lax.broadcasted_iota(jnp.int32, y.shape, 1)
        is0 = col == 0
        y0 = jnp.sum(jnp.where(is0, y, 0.0), axis=-1, keepdims=True)
        nsq = jnp.sum(jnp.where(is0, 0.0, y * y), axis=-1, keepdims=True)
        es = jnp.exp(sc_ref[0:1, 0:1])
        time = 1.0 / (1.0 + jnp.exp(-y0)) * es + 1.1
        sfac = (time * time - 1.0) / jnp.clip(nsq, 1e-8, None)
        out = jnp.where(is0, time, y * jnp.sqrt(sfac))
        o0_ref[...] = out[:, :HALF]
        o1_ref[...] = out[:, HALF:]

    if h_full is not None:
        in_arrays = (h_full,)
        in_specs = [pl.BlockSpec((BLK, D), lambda i: (i, 0))]
    else:
        in_arrays = halves
        in_specs = [pl.BlockSpec((BLK, HALF), lambda i: (i, 0)),
                    pl.BlockSpec((BLK, HALF), lambda i: (i, 0))]
    in_specs += [pl.BlockSpec((D, D), lambda i: (0, 0)),
                 pl.BlockSpec((8, D), lambda i: (0, 0)),
                 pl.BlockSpec((8, D), lambda i: (0, 0))]
    out_specs = [pl.BlockSpec((BLK, HALF), lambda i: (i, 0)),
                 pl.BlockSpec((BLK, HALF), lambda i: (i, 0))]
    return pl.pallas_call(
        body,
        grid=(grid,),
        in_specs=in_specs,
        out_specs=out_specs,
        out_shape=[jax.ShapeDtypeStruct((R, HALF), jnp.float32)] * 2,
    )(*in_arrays, Wi, b2, s2)


_CHUNK = 125
_NPHASE = 2
_PCHUNK = 40
_RPAD = 10240
_SLAB = _RPAD // NS


def _edge_agg(h0, h1, src3, dst3):
    mesh = plsc.VectorSubcoreMesh(core_axis_name="c", subcore_axis_name="s")
    z = jnp.zeros((_RPAD, HALF), jnp.float32)

    @functools.partial(
        pl.kernel,
        out_type=(jax.ShapeDtypeStruct((_RPAD, HALF), jnp.float32),
                  jax.ShapeDtypeStruct((_RPAD, HALF), jnp.float32)),
        mesh=mesh,
        scratch_types=[
            pltpu.VMEM((_PCHUNK, _CHUNK), jnp.int32),
            pltpu.VMEM((_PCHUNK, _CHUNK), jnp.int32),
            pltpu.VMEM((_CHUNK, HALF), jnp.float32),
            pltpu.VMEM((_CHUNK, HALF), jnp.float32),
            pltpu.VMEM_SHARED((_RPAD, HALF), jnp.float32),
            pltpu.SemaphoreType.DMA,
            pltpu.SemaphoreType.DMA,
        ],
    )
    def k(h0_hbm, h1_hbm, src_hbm, dst_hbm, z_hbm, o0_hbm, o1_hbm,
          src_v, dst_v, buf0, buf1, acc, semA, semB):
        c = lax.axis_index("c")
        s = lax.axis_index("s")

        def run(h_hbm, o_hbm):
            pltpu.sync_copy(z_hbm.at[pl.ds(s * _SLAB, _SLAB)],
                            acc.at[pl.ds(s * _SLAB, _SLAB)])
            plsc.subcore_barrier()
            for p in range(_NPHASE):
                pltpu.sync_copy(src_hbm.at[s, p], src_v)
                pltpu.sync_copy(dst_hbm.at[s, p], dst_v)
                pltpu.make_async_copy(h_hbm.at[src_v.at[0]], buf0, semA).start()

                def body(kk, carry):
                    j0 = 2 * kk
                    pltpu.make_async_copy(h_hbm.at[src_v.at[j0 + 1]], buf1, semB).start()
                    pltpu.make_async_copy(h_hbm.at[src_v.at[j0]], buf0, semA).wait()
                    pltpu.sync_copy(buf0, acc.at[dst_v.at[j0]], add=True)
                    j2 = lax.rem(j0 + 2, _PCHUNK)
                    pltpu.make_async_copy(h_hbm.at[src_v.at[j2]], buf0, semA).start()
                    pltpu.make_async_copy(h_hbm.at[src_v.at[j0 + 1]], buf1, semB).wait()
                    pltpu.sync_copy(buf1, acc.at[dst_v.at[j0 + 1]], add=True)
                    return carry

                lax.fori_loop(0, _PCHUNK // 2, body, 0)
                pltpu.make_async_copy(h_hbm.at[src_v.at[0]], buf0, semA).wait()
            plsc.subcore_barrier()
            pltpu.sync_copy(acc.at[pl.ds(s * _SLAB, _SLAB)],
                            o_hbm.at[pl.ds(s * _SLAB, _SLAB)])

        @pl.when(c == 0)
        def _():
            run(h0_hbm, o0_hbm)

        @pl.when(c == 1)
        def _():
            run(h1_hbm, o1_hbm)

    return k(h0, h1, src3, dst3, z)


_FBLK = 500
_FTREES = 5
_FGRID = N_NODES // _FBLK


def _finalize(s0, s1, scope):
    s0_3 = s0[:N_NODES].reshape(_FGRID, _FBLK, HALF)
    s1_3 = s1[:N_NODES].reshape(_FGRID, _FBLK, HALF)
    scope3 = jnp.broadcast_to(
        scope.astype(jnp.float32).reshape(_FGRID, _FTREES, 1),
        (_FGRID, _FTREES, D))

    def body(s0_ref, s1_ref, sc_ref, h_ref, t_ref):
        h = jnp.concatenate([s0_ref[0], s1_ref[0]], axis=-1)
        col = lax.broadcasted_iota(jnp.int32, h.shape, 1)
        is0 = col == 0
        t = jnp.sum(jnp.where(is0, h, 0.0), axis=-1, keepdims=True)
        nsq = jnp.sum(jnp.where(is0, 0.0, h * h), axis=-1, keepdims=True)
        denom = jnp.sqrt(jnp.clip(jnp.abs(t * t - nsq), 1e-8, None))
        hn = h / denom
        h_ref[...] = hn[None]
        ti = lax.broadcasted_iota(jnp.int32, (_FTREES, _FBLK), 0)
        ri = lax.broadcasted_iota(jnp.int32, (_FTREES, _FBLK), 1)
        mask = (ri // TREE == ti).astype(jnp.float32)
        sums = jnp.dot(mask, hn, precision=lax.Precision.HIGHEST,
                       preferred_element_type=jnp.float32)
        ave = sums / sc_ref[0]
        col5 = lax.broadcasted_iota(jnp.int32, ave.shape, 1)
        is05 = col5 == 0
        t5 = jnp.sum(jnp.where(is05, ave, 0.0), axis=-1, keepdims=True)
        nsq5 = jnp.sum(jnp.where(is05, 0.0, ave * ave), axis=-1, keepdims=True)
        denom5 = jnp.sqrt(jnp.clip(jnp.abs(t5 * t5 - nsq5), 1e-8, None))
        t_ref[...] = (ave / denom5)[None]

    h3, tree3 = pl.pallas_call(
        body,
        grid=(_FGRID,),
        in_specs=[pl.BlockSpec((1, _FBLK, HALF), lambda i: (i, 0, 0)),
                  pl.BlockSpec((1, _FBLK, HALF), lambda i: (i, 0, 0)),
                  pl.BlockSpec((1, _FTREES, D), lambda i: (i, 0, 0))],
        out_specs=[pl.BlockSpec((1, _FBLK, D), lambda i: (i, 0, 0)),
                   pl.BlockSpec((1, _FTREES, D), lambda i: (i, 0, 0))],
        out_shape=[jax.ShapeDtypeStruct((_FGRID, _FBLK, D), jnp.float32),
                   jax.ShapeDtypeStruct((_FGRID, _FTREES, D), jnp.float32)],
    )(s0_3, s1_3, scope3)
    return tree3.reshape(N_TREES, D), h3.reshape(N_NODES, D)


def kernel(x, edge_index, scope, emb, W, b, scale):
    x = x.astype(jnp.int32)
    src = edge_index[0].astype(jnp.int32)
    dst = edge_index[1].astype(jnp.int32)
    src3 = src.reshape(NS, _NPHASE, _PCHUNK, _CHUNK)
    dst3 = dst.reshape(NS, _NPHASE, _PCHUNK, _CHUNK)

    h_emb = _embed_gather(x, emb)
    h0, h1 = _lorentz_layer(h_emb, None, W[0], b[0], scale[0], nonlin=False)
    s0, s1 = _edge_agg(h0, h1, src3, dst3)
    h0, h1 = _lorentz_layer(None, (s0, s1), W[1], b[1], scale[1], nonlin=True)
    s0, s1 = _edge_agg(h0, h1, src3, dst3)
    h0, h1 = _lorentz_layer(None, (s0, s1), W[2], b[2], scale[2], nonlin=True)
    s0, s1 = _edge_agg(h0, h1, src3, dst3)
    tree_vecs, h = _finalize(s0, s1, scope)
    return (tree_vecs, h)


# speedup vs baseline: 6.4599x; 1.0212x over previous
import functools

import jax
import jax.numpy as jnp
from jax import lax
from jax.experimental import pallas as pl
from jax.experimental.pallas import tpu as pltpu
from jax.experimental.pallas import tpu_sc as plsc

N_NODES = 10000
N_EDGES = 160000
D = 256
HALF = 128
VOCAB = 780
N_TREES = 100
TREE = 100

NC = 2
NS = 16
NW = NC * NS


_EPAD = 10240
_ECHUNK = 80
_ECHUNKS = 4


def _embed_gather(x, emb):
    xp = jnp.concatenate([x, jnp.zeros((_EPAD - N_NODES,), jnp.int32)])
    x3 = xp.reshape(NW, _ECHUNKS, _ECHUNK)
    mesh = plsc.VectorSubcoreMesh(core_axis_name="c", subcore_axis_name="s")

    @functools.partial(
        pl.kernel,
        out_type=jax.ShapeDtypeStruct((_EPAD, D), jnp.float32),
        mesh=mesh,
        scratch_types=[
            pltpu.VMEM((_ECHUNKS, _ECHUNK), jnp.int32),
            pltpu.VMEM((_ECHUNK, D), jnp.float32),
            pltpu.VMEM((_ECHUNK, D), jnp.float32),
            pltpu.SemaphoreType.DMA,
            pltpu.SemaphoreType.DMA,
        ],
    )
    def k(emb_hbm, x_hbm, out_hbm, idx_v, rows0, rows1, semA, semB):
        c = lax.axis_index("c")
        s = lax.axis_index("s")
        w = s * NC + c
        pltpu.sync_copy(x_hbm.at[w], idx_v)
        base = w * (_ECHUNKS * _ECHUNK)
        pltpu.make_async_copy(emb_hbm.at[idx_v.at[0]], rows0, semA).start()
        pltpu.make_async_copy(emb_hbm.at[idx_v.at[1]], rows1, semB).start()
        for j in range(_ECHUNKS):
            buf = rows0 if j % 2 == 0 else rows1
            sem = semA if j % 2 == 0 else semB
            pltpu.make_async_copy(emb_hbm.at[idx_v.at[j]], buf, sem).wait()
            pltpu.sync_copy(buf, out_hbm.at[pl.ds(base + j * _ECHUNK, _ECHUNK)])
            if j + 2 < _ECHUNKS:
                pltpu.make_async_copy(emb_hbm.at[idx_v.at[j + 2]], buf, sem).start()

    return k(emb, x3)


def _lorentz_layer(h_full, halves, Wi, bi, scalei, nonlin):
    if h_full is not None:
        R = h_full.shape[0]
    else:
        R = halves[0].shape[0]
    BLK = 1024 if R % 1024 == 0 else 1000
    grid = R // BLK
    b2 = jnp.broadcast_to(bi.reshape(1, D), (8, D))
    s2 = jnp.broadcast_to(scalei.reshape(1, 1), (8, D))

    def body(*refs):
        if h_full is not None:
            h_ref, w_ref, b_ref, sc_ref, o0_ref, o1_ref = refs
            h = h_ref[...]
        else:
            s0_ref, s1_ref, w_ref, b_ref, sc_ref, o0_ref, o1_ref = refs
            h = jnp.concatenate([s0_ref[...], s1_ref[...]], axis=-1)
            col = lax.broadcasted_iota(jnp.int32, h.shape, 1)
            is0 = col == 0
            t = jnp.sum(jnp.where(is0, h, 0.0), axis=-1, keepdims=True)
            nsq = jnp.sum(jnp.where(is0, 0.0, h * h), axis=-1, keepdims=True)
            denom = jnp.sqrt(jnp.clip(jnp.abs(t * t - nsq), 1e-8, None))
            h = h / denom
        if nonlin:
            h = jnp.maximum(h, 0.0)
        y = lax.dot_general(h, w_ref[...], (((1,), (1,)), ((), ())),
                            precision=lax.Precision.HIGHEST,
                            preferred_element_type=jnp.float32)
        y = y + b_ref[0:1, :]
        col = lax.broadcasted_iota(jnp.int32, y.shape, 1)
        is0 = col == 0
        y0 = jnp.sum(jnp.where(is0, y, 0.0), axis=-1, keepdims=True)
        nsq = jnp.sum(jnp.where(is0, 0.0, y * y), axis=-1, keepdims=True)
        es = jnp.exp(sc_ref[0:1, 0:1])
        time = 1.0 / (1.0 + jnp.exp(-y0)) * es + 1.1
        sfac = (time * time - 1.0) / jnp.clip(nsq, 1e-8, None)
        out = jnp.where(is0, time, y * jnp.sqrt(sfac))
        o0_ref[...] = out[:, :HALF]
        o1_ref[...] = out[:, HALF:]

    if h_full is not None:
        in_arrays = (h_full,)
        in_specs = [pl.BlockSpec((BLK, D), lambda i: (i, 0))]
    else:
        in_arrays = halves
        in_specs = [pl.BlockSpec((BLK, HALF), lambda i: (i, 0)),
                    pl.BlockSpec((BLK, HALF), lambda i: (i, 0))]
    in_specs += [pl.BlockSpec((D, D), lambda i: (0, 0)),
                 pl.BlockSpec((8, D), lambda i: (0, 0)),
                 pl.BlockSpec((8, D), lambda i: (0, 0))]
    out_specs = [pl.BlockSpec((BLK, HALF), lambda i: (i, 0)),
                 pl.BlockSpec((BLK, HALF), lambda i: (i, 0))]
    return pl.pallas_call(
        body,
        grid=(grid,),
        in_specs=in_specs,
        out_specs=out_specs,
        out_shape=[jax.ShapeDtypeStruct((R, HALF), jnp.float32)] * 2,
    )(*in_arrays, Wi, b2, s2)


_CHUNK = 125
_NPHASE = 2
_PCHUNK = 40
_RPAD = 10240
_SLAB = _RPAD // NS


def _edge_agg(h0, h1, src3, dst3):
    mesh = plsc.VectorSubcoreMesh(core_axis_name="c", subcore_axis_name="s")

    @functools.partial(
        pl.kernel,
        out_type=(jax.ShapeDtypeStruct((_RPAD, HALF), jnp.float32),
                  jax.ShapeDtypeStruct((_RPAD, HALF), jnp.float32)),
        mesh=mesh,
        scratch_types=[
            pltpu.VMEM((_PCHUNK, _CHUNK), jnp.int32),
            pltpu.VMEM((_PCHUNK, _CHUNK), jnp.int32),
            pltpu.VMEM((_CHUNK, HALF), jnp.float32),
            pltpu.VMEM((_CHUNK, HALF), jnp.float32),
            pltpu.VMEM_SHARED((_RPAD, HALF), jnp.float32),
            pltpu.SemaphoreType.DMA,
            pltpu.SemaphoreType.DMA,
        ],
    )
    def k(h0_hbm, h1_hbm, src_hbm, dst_hbm, o0_hbm, o1_hbm,
          src_v, dst_v, buf0, buf1, acc, semA, semB):
        c = lax.axis_index("c")
        s = lax.axis_index("s")

        def run(h_hbm, o_hbm):
            zv = jnp.zeros((16,), jnp.float32)

            def zrow(r, carry):
                for gg in range(HALF // 16):
                    buf0[r, pl.ds(gg * 16, 16)] = zv
                return carry

            lax.fori_loop(0, 80, zrow, 0)
            for q in range(_SLAB // 80):
                pltpu.sync_copy(buf0.at[pl.ds(0, 80)],
                                acc.at[pl.ds(s * _SLAB + q * 80, 80)])
            plsc.subcore_barrier()
            for p in range(_NPHASE):
                pltpu.sync_copy(src_hbm.at[s, p], src_v)
                pltpu.sync_copy(dst_hbm.at[s, p], dst_v)
                pltpu.make_async_copy(h_hbm.at[src_v.at[0]], buf0, semA).start()

                def body(kk, carry):
                    j0 = 2 * kk
                    pltpu.make_async_copy(h_hbm.at[src_v.at[j0 + 1]], buf1, semB).start()
                    pltpu.make_async_copy(h_hbm.at[src_v.at[j0]], buf0, semA).wait()
                    pltpu.sync_copy(buf0, acc.at[dst_v.at[j0]], add=True)
                    j2 = lax.rem(j0 + 2, _PCHUNK)
                    pltpu.make_async_copy(h_hbm.at[src_v.at[j2]], buf0, semA).start()
                    pltpu.make_async_copy(h_hbm.at[src_v.at[j0 + 1]], buf1, semB).wait()
                    pltpu.sync_copy(buf1, acc.at[dst_v.at[j0 + 1]], add=True)
                    return carry

                lax.fori_loop(0, _PCHUNK // 2, body, 0)
                pltpu.make_async_copy(h_hbm.at[src_v.at[0]], buf0, semA).wait()
            plsc.subcore_barrier()
            pltpu.sync_copy(acc.at[pl.ds(s * _SLAB, _SLAB)],
                            o_hbm.at[pl.ds(s * _SLAB, _SLAB)])

        @pl.when(c == 0)
        def _():
            run(h0_hbm, o0_hbm)

        @pl.when(c == 1)
        def _():
            run(h1_hbm, o1_hbm)

    return k(h0, h1, src3, dst3)


_FBLK = 500
_FTREES = 5
_FGRID = N_NODES // _FBLK


def _finalize(s0, s1, scope):
    s0_3 = s0[:N_NODES].reshape(_FGRID, _FBLK, HALF)
    s1_3 = s1[:N_NODES].reshape(_FGRID, _FBLK, HALF)
    scope3 = jnp.broadcast_to(
        scope.astype(jnp.float32).reshape(_FGRID, _FTREES, 1),
        (_FGRID, _FTREES, D))

    def body(s0_ref, s1_ref, sc_ref, h_ref, t_ref):
        h = jnp.concatenate([s0_ref[0], s1_ref[0]], axis=-1)
        col = lax.broadcasted_iota(jnp.int32, h.shape, 1)
        is0 = col == 0
        t = jnp.sum(jnp.where(is0, h, 0.0), axis=-1, keepdims=True)
        nsq = jnp.sum(jnp.where(is0, 0.0, h * h), axis=-1, keepdims=True)
        denom = jnp.sqrt(jnp.clip(jnp.abs(t * t - nsq), 1e-8, None))
        hn = h / denom
        h_ref[...] = hn[None]
        ti = lax.broadcasted_iota(jnp.int32, (_FTREES, _FBLK), 0)
        ri = lax.broadcasted_iota(jnp.int32, (_FTREES, _FBLK), 1)
        mask = (ri // TREE == ti).astype(jnp.float32)
        sums = jnp.dot(mask, hn, precision=lax.Precision.HIGHEST,
                       preferred_element_type=jnp.float32)
        ave = sums / sc_ref[0]
        col5 = lax.broadcasted_iota(jnp.int32, ave.shape, 1)
        is05 = col5 == 0
        t5 = jnp.sum(jnp.where(is05, ave, 0.0), axis=-1, keepdims=True)
        nsq5 = jnp.sum(jnp.where(is05, 0.0, ave * ave), axis=-1, keepdims=True)
        denom5 = jnp.sqrt(jnp.clip(jnp.abs(t5 * t5 - nsq5), 1e-8, None))
        t_ref[...] = (ave / denom5)[None]

    h3, tree3 = pl.pallas_call(
        body,
        grid=(_FGRID,),
        in_specs=[pl.BlockSpec((1, _FBLK, HALF), lambda i: (i, 0, 0)),
                  pl.BlockSpec((1, _FBLK, HALF), lambda i: (i, 0, 0)),
                  pl.BlockSpec((1, _FTREES, D), lambda i: (i, 0, 0))],
        out_specs=[pl.BlockSpec((1, _FBLK, D), lambda i: (i, 0, 0)),
                   pl.BlockSpec((1, _FTREES, D), lambda i: (i, 0, 0))],
        out_shape=[jax.ShapeDtypeStruct((_FGRID, _FBLK, D), jnp.float32),
                   jax.ShapeDtypeStruct((_FGRID, _FTREES, D), jnp.float32)],
    )(s0_3, s1_3, scope3)
    return tree3.reshape(N_TREES, D), h3.reshape(N_NODES, D)


def kernel(x, edge_index, scope, emb, W, b, scale):
    x = x.astype(jnp.int32)
    src = edge_index[0].astype(jnp.int32)
    dst = edge_index[1].astype(jnp.int32)
    src3 = src.reshape(NS, _NPHASE, _PCHUNK, _CHUNK)
    dst3 = dst.reshape(NS, _NPHASE, _PCHUNK, _CHUNK)

    h_emb = _embed_gather(x, emb)
    h0, h1 = _lorentz_layer(h_emb, None, W[0], b[0], scale[0], nonlin=False)
    s0, s1 = _edge_agg(h0, h1, src3, dst3)
    h0, h1 = _lorentz_layer(None, (s0, s1), W[1], b[1], scale[1], nonlin=True)
    s0, s1 = _edge_agg(h0, h1, src3, dst3)
    h0, h1 = _lorentz_layer(None, (s0, s1), W[2], b[2], scale[2], nonlin=True)
    s0, s1 = _edge_agg(h0, h1, src3, dst3)
    tree_vecs, h = _finalize(s0, s1, scope)
    return (tree_vecs, h)
